# baseline (speedup 1.0000x reference)
import jax
import jax.numpy as jnp
from jax import lax
from jax.experimental import pallas as pl
from jax.experimental.pallas import tpu as pltpu
from jax.experimental.pallas import tpu_sc as plsc

N_NODES = 10000
IN_CH = 128
HID_CH = 128
N_EDGES = 320000

_NC = 2
_NS = 16
_NW = _NC * _NS
_EPW = N_EDGES // _NW
_C = 80
_NCHUNK = _EPW // _C
_RPT = 1000
_NDRAIN = N_NODES // _RPT


def _encode_body(x_ref, w_ref, asr_ref, adr_ref, h_ref, als_ref, ald_ref,
                 m_ref, mx_ref):
    h = jnp.dot(x_ref[...], w_ref[...], preferred_element_type=jnp.float32)
    h_ref[...] = h
    a_s = jnp.sum(h * asr_ref[...], axis=1, keepdims=True)
    a_d = jnp.sum(h * adr_ref[...], axis=1, keepdims=True)
    als_ref[...] = jnp.broadcast_to(a_s, a_s.shape[:1] + (16,))
    ald_ref[...] = jnp.broadcast_to(a_d, a_d.shape[:1] + (16,))

    i = pl.program_id(0)

    @pl.when(i == 0)
    def _():
        mx_ref[...] = jnp.full((2, 128), -jnp.inf, jnp.float32)

    bs = jnp.max(a_s)
    bd = jnp.max(a_d)
    upd = jnp.concatenate([jnp.full((1, 128), bs, jnp.float32),
                           jnp.full((1, 128), bd, jnp.float32)], axis=0)
    mx_ref[...] = jnp.maximum(mx_ref[...], upd)

    @pl.when(i == pl.num_programs(0) - 1)
    def _():
        mb = mx_ref[0:1, :] + mx_ref[1:2, :]
        m = jnp.where(mb > 0, mb, 0.2 * mb)
        m_ref[...] = m[:, :16]


def _encode(x, w_enc, a_src, a_dst):
    blk = 1000
    grid = (N_NODES // blk,)
    return pl.pallas_call(
        _encode_body,
        grid=grid,
        in_specs=[
            pl.BlockSpec((blk, IN_CH), lambda i: (i, 0)),
            pl.BlockSpec((IN_CH, HID_CH), lambda i: (0, 0)),
            pl.BlockSpec((1, HID_CH), lambda i: (0, 0)),
            pl.BlockSpec((1, HID_CH), lambda i: (0, 0)),
        ],
        out_specs=[
            pl.BlockSpec((blk, HID_CH), lambda i: (i, 0)),
            pl.BlockSpec((blk, 16), lambda i: (i, 0)),
            pl.BlockSpec((blk, 16), lambda i: (i, 0)),
            pl.BlockSpec((1, 16), lambda i: (0, 0)),
            pl.BlockSpec((2, 128), lambda i: (0, 0)),
        ],
        out_shape=[
            jax.ShapeDtypeStruct((N_NODES, HID_CH), jnp.float32),
            jax.ShapeDtypeStruct((N_NODES, 16), jnp.float32),
            jax.ShapeDtypeStruct((N_NODES, 16), jnp.float32),
            jax.ShapeDtypeStruct((1, 16), jnp.float32),
            jax.ShapeDtypeStruct((2, 128), jnp.float32),
        ],
    )(x, w_enc, a_src.reshape(1, HID_CH), a_dst.reshape(1, HID_CH))


def _sc_body(h_hbm, ei_hbm, als_hbm, ald_hbm, m_hbm,
             num_out, den_out,
             m_v, ei3, asr2, adr2, rows2, wden2, w_v,
             num_sh, den_sh, sem_i, sem_g, sem_a, sem_s):
    cid = lax.axis_index("c")
    sid = lax.axis_index("s")
    wid = cid * _NS + sid

    pltpu.sync_copy(m_hbm, m_v)

    @pl.loop(0, _C)
    def _zrow(r):
        for c in range(8):
            rows2[0, r, pl.ds(c * 16, 16)] = jnp.zeros((16,), jnp.float32)
        wden2[0, r, pl.ds(0, 16)] = jnp.zeros((16,), jnp.float32)
        wden2[1, r, pl.ds(0, 16)] = jnp.zeros((16,), jnp.float32)

    @pl.when(sid < _NDRAIN)
    def _():
        base_r = sid * _RPT
        for k in range(12):
            pltpu.sync_copy(rows2.at[0], num_sh.at[pl.ds(base_r + k * _C, _C)])
            pltpu.sync_copy(wden2.at[0], den_sh.at[pl.ds(base_r + k * _C, _C)])
        pltpu.sync_copy(rows2.at[0, pl.ds(0, 40)],
                        num_sh.at[pl.ds(base_r + 960, 40)])
        pltpu.sync_copy(wden2.at[0, pl.ds(0, 40)],
                        den_sh.at[pl.ds(base_r + 960, 40)])

    plsc.subcore_barrier()

    m16 = m_v[0, :]
    base = wid * _EPW

    def idx_start(g, slot):
        off = base + g * _C
        pltpu.async_copy(ei_hbm.at[:, pl.ds(off, _C)], ei3.at[slot], sem_i)

    def idx_wait(slot):
        pltpu.make_async_copy(ei_hbm.at[:, pl.ds(0, _C)], ei3.at[slot],
                              sem_i).wait()

    _H = _C // 2

    def gath_start(slot, b):
        for k in range(2):
            sl = pl.ds(k * _H, _H)
            pltpu.async_copy(h_hbm.at[ei3.at[slot, 0, sl]],
                             rows2.at[b, sl], sem_g.at[b, k])
        pltpu.async_copy(als_hbm.at[ei3.at[slot, 0]], asr2.at[b], sem_a.at[b])
        pltpu.async_copy(ald_hbm.at[ei3.at[slot, 1]], adr2.at[b], sem_a.at[b])

    def rows_wait(slot, b, k):
        sl = pl.ds(k * _H, _H)
        pltpu.make_async_copy(h_hbm.at[ei3.at[slot, 0, sl]],
                              rows2.at[b, sl], sem_g.at[b, k]).wait()

    def alpha_wait(slot, b):
        pltpu.make_async_copy(als_hbm.at[ei3.at[slot, 0]], asr2.at[b],
                              sem_a.at[b]).wait()
        pltpu.make_async_copy(ald_hbm.at[ei3.at[slot, 1]], adr2.at[b],
                              sem_a.at[b]).wait()

    def scat_start(slot, b):
        pltpu.async_copy(rows2.at[b], num_sh.at[ei3.at[slot, 1]],
                         sem_s.at[b], add=True)
        pltpu.async_copy(wden2.at[b], den_sh.at[ei3.at[slot, 1]],
                         sem_s.at[b], add=True)

    def scat_wait(slot, b):
        pltpu.make_async_copy(rows2.at[b], num_sh.at[ei3.at[slot, 1]],
                              sem_s.at[b]).wait()
        pltpu.make_async_copy(wden2.at[b], den_sh.at[ei3.at[slot, 1]],
                              sem_s.at[b]).wait()

    idx_start(0, 0)
    idx_wait(0)
    idx_start(1, 1)
    gath_start(0, 0)

    z16 = jnp.zeros((16,), jnp.int32)
    i16 = lax.iota(jnp.int32, 16)

    @pl.loop(0, _NCHUNK)
    def _chunk(g):
        p = lax.rem(g, 2)
        slot = lax.rem(g, 3)

        @pl.when(g > 0)
        def _():
            scat_wait(lax.rem(g + 2, 3), 1 - p)

        @pl.when(g < _NCHUNK - 1)
        def _():
            idx_wait(lax.rem(g + 1, 3))
            gath_start(lax.rem(g + 1, 3), 1 - p)

        @pl.when(g < _NCHUNK - 2)
        def _():
            idx_start(g + 2, lax.rem(g + 2, 3))

        alpha_wait(slot, p)
        for j in range(_C // 16):
            r16 = i16 + (j * 16)
            e = (plsc.load_gather(asr2.at[p], [r16, z16])
                 + plsc.load_gather(adr2.at[p], [r16, z16]))
            e = jnp.where(e > 0, e, e * 0.2)
            w16 = jnp.exp(e - m16)
            w_v[pl.ds(j * 16, 16)] = w16
            plsc.store_scatter(wden2.at[p], [r16, z16], w16)

        rows_wait(slot, p, 0)

        @plsc.parallel_loop(0, _C // 2, 1, unroll=8)
        def _row_lo(r):
            wb = plsc.load_gather(w_v, [jnp.full((16,), r, jnp.int32)])
            for c in range(8):
                sl = pl.ds(c * 16, 16)
                rows2[p, r, sl] = rows2[p, r, sl] * wb

        rows_wait(slot, p, 1)

        @plsc.parallel_loop(_C // 2, _C, 1, unroll=8)
        def _row_hi(r):
            wb = plsc.load_gather(w_v, [jnp.full((16,), r, jnp.int32)])
            for c in range(8):
                sl = pl.ds(c * 16, 16)
                rows2[p, r, sl] = rows2[p, r, sl] * wb

        scat_start(slot, p)

    scat_wait(lax.rem(_NCHUNK - 1, 3), lax.rem(_NCHUNK - 1, 2))
    plsc.subcore_barrier()

    @pl.when(sid < _NDRAIN)
    def _():
        r0 = sid * _RPT
        pltpu.sync_copy(num_sh.at[pl.ds(r0, _RPT)],
                        num_out.at[cid, pl.ds(r0, _RPT)])
        pltpu.sync_copy(den_sh.at[pl.ds(r0, _RPT)],
                        den_out.at[cid, pl.ds(r0, _RPT)])


def _sc_edge_pass(h, edge_index, alpha_s, alpha_d, m_vec):
    mesh = plsc.VectorSubcoreMesh(core_axis_name="c", subcore_axis_name="s")
    f32 = jnp.float32
    k = pl.kernel(
        _sc_body,
        compiler_params=pltpu.CompilerParams(needs_layout_passes=False,
                                             use_tc_tiling_on_sc=False),
        out_type=(
            jax.ShapeDtypeStruct((_NC, N_NODES, HID_CH), f32),
            jax.ShapeDtypeStruct((_NC, N_NODES, 16), f32),
        ),
        mesh=mesh,
        scratch_types=[
            pltpu.VMEM((1, 16), f32),
            pltpu.VMEM((3, 2, _C), jnp.int32),
            pltpu.VMEM((2, _C, 16), f32),
            pltpu.VMEM((2, _C, 16), f32),
            pltpu.VMEM((2, _C, HID_CH), f32),
            pltpu.VMEM((2, _C, 16), f32),
            pltpu.VMEM((_C,), f32),
            pltpu.VMEM_SHARED((N_NODES, HID_CH), f32),
            pltpu.VMEM_SHARED((N_NODES, 16), f32),
            pltpu.SemaphoreType.DMA,
            pltpu.SemaphoreType.DMA((2, 2)),
            pltpu.SemaphoreType.DMA((2,)),
            pltpu.SemaphoreType.DMA((2,)),
        ],
    )
    return k(h, edge_index, alpha_s, alpha_d, m_vec)


def _decode_body(n0_ref, n1_ref, d0_ref, d1_ref, be_ref, wd_ref, bd_ref,
                 rec_ref, z_ref):
    num = n0_ref[0] + n1_ref[0]
    den = d0_ref[0, :, 0:1] + d1_ref[0, :, 0:1]
    z = num / (den + 1e-16) + be_ref[...]
    z_ref[...] = z
    rec_ref[...] = jnp.tanh(
        jnp.dot(z, wd_ref[...], preferred_element_type=jnp.float32)
        + bd_ref[...])


def _decode(num, den, b_enc, w_dec, b_dec):
    blk = 1000
    grid = (N_NODES // blk,)
    return pl.pallas_call(
        _decode_body,
        grid=grid,
        in_specs=[
            pl.BlockSpec((1, blk, HID_CH), lambda i: (0, i, 0)),
            pl.BlockSpec((1, blk, HID_CH), lambda i: (1, i, 0)),
            pl.BlockSpec((1, blk, 16), lambda i: (0, i, 0)),
            pl.BlockSpec((1, blk, 16), lambda i: (1, i, 0)),
            pl.BlockSpec((1, HID_CH), lambda i: (0, 0)),
            pl.BlockSpec((HID_CH, IN_CH), lambda i: (0, 0)),
            pl.BlockSpec((1, IN_CH), lambda i: (0, 0)),
        ],
        out_specs=[
            pl.BlockSpec((blk, IN_CH), lambda i: (i, 0)),
            pl.BlockSpec((blk, HID_CH), lambda i: (i, 0)),
        ],
        out_shape=[
            jax.ShapeDtypeStruct((N_NODES, IN_CH), jnp.float32),
            jax.ShapeDtypeStruct((N_NODES, HID_CH), jnp.float32),
        ],
    )(num, num, den, den, b_enc.reshape(1, HID_CH), w_dec,
      b_dec.reshape(1, IN_CH))


def kernel(x, edge_index, W_enc, a_src, a_dst, b_enc, W_dec, b_dec):
    h, als, ald, m_vec, _ = _encode(x, W_enc, a_src, a_dst)
    num, den = _sc_edge_pass(h, edge_index, als, ald, m_vec)
    recon, z = _decode(num, den, b_enc, W_dec, b_dec)
    return (recon, z)


# speedup vs baseline: 1.0220x; 1.0220x over previous
import jax
import jax.numpy as jnp
from jax import lax
from jax.experimental import pallas as pl
from jax.experimental.pallas import tpu as pltpu
from jax.experimental.pallas import tpu_sc as plsc

N_NODES = 10000
IN_CH = 128
HID_CH = 128
N_EDGES = 320000

_NC = 2
_NS = 16
_NW = _NC * _NS
_EPW = N_EDGES // _NW
_C = 80
_NCHUNK = _EPW // _C
_RPT = 1000
_NDRAIN = N_NODES // _RPT


def _encode_body(x_ref, w_ref, asr_ref, adr_ref, h_ref, als_ref, ald_ref,
                 m_ref, mx_ref):
    h = jnp.dot(x_ref[...], w_ref[...], preferred_element_type=jnp.float32)
    h_ref[...] = h
    a_s = jnp.sum(h * asr_ref[...], axis=1, keepdims=True)
    a_d = jnp.sum(h * adr_ref[...], axis=1, keepdims=True)
    als_ref[...] = jnp.broadcast_to(a_s, a_s.shape[:1] + (16,))
    ald_ref[...] = jnp.broadcast_to(a_d, a_d.shape[:1] + (16,))

    i = pl.program_id(0)

    @pl.when(i == 0)
    def _():
        mx_ref[...] = jnp.full((2, 128), -jnp.inf, jnp.float32)

    bs = jnp.max(a_s)
    bd = jnp.max(a_d)
    upd = jnp.concatenate([jnp.full((1, 128), bs, jnp.float32),
                           jnp.full((1, 128), bd, jnp.float32)], axis=0)
    mx_ref[...] = jnp.maximum(mx_ref[...], upd)

    @pl.when(i == pl.num_programs(0) - 1)
    def _():
        mb = mx_ref[0:1, :] + mx_ref[1:2, :]
        m = jnp.where(mb > 0, mb, 0.2 * mb)
        m_ref[...] = m[:, :16]


def _encode(x, w_enc, a_src, a_dst):
    blk = 1000
    grid = (N_NODES // blk,)
    return pl.pallas_call(
        _encode_body,
        grid=grid,
        in_specs=[
            pl.BlockSpec((blk, IN_CH), lambda i: (i, 0)),
            pl.BlockSpec((IN_CH, HID_CH), lambda i: (0, 0)),
            pl.BlockSpec((1, HID_CH), lambda i: (0, 0)),
            pl.BlockSpec((1, HID_CH), lambda i: (0, 0)),
        ],
        out_specs=[
            pl.BlockSpec((blk, HID_CH), lambda i: (i, 0)),
            pl.BlockSpec((blk, 16), lambda i: (i, 0)),
            pl.BlockSpec((blk, 16), lambda i: (i, 0)),
            pl.BlockSpec((1, 16), lambda i: (0, 0)),
            pl.BlockSpec((2, 128), lambda i: (0, 0)),
        ],
        out_shape=[
            jax.ShapeDtypeStruct((N_NODES, HID_CH), jnp.float32),
            jax.ShapeDtypeStruct((N_NODES, 16), jnp.float32),
            jax.ShapeDtypeStruct((N_NODES, 16), jnp.float32),
            jax.ShapeDtypeStruct((1, 16), jnp.float32),
            jax.ShapeDtypeStruct((2, 128), jnp.float32),
        ],
    )(x, w_enc, a_src.reshape(1, HID_CH), a_dst.reshape(1, HID_CH))


def _sc_body(h_hbm, ei_hbm, als_hbm, ald_hbm, m_hbm,
             num_out, den_out,
             m_v, ei3, asr2, adr2, rows2, wden2, w_v,
             num_sh, den_sh, sem_i, sem_g, sem_a, sem_s):
    cid = lax.axis_index("c")
    sid = lax.axis_index("s")
    wid = cid * _NS + sid

    pltpu.sync_copy(m_hbm, m_v)

    @pl.loop(0, _C)
    def _zrow(r):
        for c in range(8):
            rows2[0, r, pl.ds(c * 16, 16)] = jnp.zeros((16,), jnp.float32)
        wden2[0, r, pl.ds(0, 16)] = jnp.zeros((16,), jnp.float32)
        wden2[1, r, pl.ds(0, 16)] = jnp.zeros((16,), jnp.float32)

    @pl.when(sid < _NDRAIN)
    def _():
        base_r = sid * _RPT
        for k in range(12):
            pltpu.async_copy(rows2.at[0],
                             num_sh.at[pl.ds(base_r + k * _C, _C)], sem_i)
            pltpu.async_copy(wden2.at[0],
                             den_sh.at[pl.ds(base_r + k * _C, _C)], sem_i)
        pltpu.async_copy(rows2.at[0, pl.ds(0, 40)],
                         num_sh.at[pl.ds(base_r + 960, 40)], sem_i)
        pltpu.async_copy(wden2.at[0, pl.ds(0, 40)],
                         den_sh.at[pl.ds(base_r + 960, 40)], sem_i)
        for k in range(12):
            pltpu.make_async_copy(
                rows2.at[0], num_sh.at[pl.ds(base_r + k * _C, _C)],
                sem_i).wait()
            pltpu.make_async_copy(
                wden2.at[0], den_sh.at[pl.ds(base_r + k * _C, _C)],
                sem_i).wait()
        pltpu.make_async_copy(rows2.at[0, pl.ds(0, 40)],
                              num_sh.at[pl.ds(base_r + 960, 40)],
                              sem_i).wait()
        pltpu.make_async_copy(wden2.at[0, pl.ds(0, 40)],
                              den_sh.at[pl.ds(base_r + 960, 40)],
                              sem_i).wait()

    plsc.subcore_barrier()

    m16 = m_v[0, :]
    base = wid * _EPW

    def idx_start(g, slot):
        off = base + g * _C
        pltpu.async_copy(ei_hbm.at[:, pl.ds(off, _C)], ei3.at[slot], sem_i)

    def idx_wait(slot):
        pltpu.make_async_copy(ei_hbm.at[:, pl.ds(0, _C)], ei3.at[slot],
                              sem_i).wait()

    def gath_start(slot, b):
        pltpu.async_copy(h_hbm.at[ei3.at[slot, 0]], rows2.at[b], sem_g.at[b])
        pltpu.async_copy(als_hbm.at[ei3.at[slot, 0]], asr2.at[b], sem_a.at[b])
        pltpu.async_copy(ald_hbm.at[ei3.at[slot, 1]], adr2.at[b], sem_a.at[b])

    def rows_wait(slot, b):
        pltpu.make_async_copy(h_hbm.at[ei3.at[slot, 0]], rows2.at[b],
                              sem_g.at[b]).wait()

    def alpha_wait(slot, b):
        pltpu.make_async_copy(als_hbm.at[ei3.at[slot, 0]], asr2.at[b],
                              sem_a.at[b]).wait()
        pltpu.make_async_copy(ald_hbm.at[ei3.at[slot, 1]], adr2.at[b],
                              sem_a.at[b]).wait()

    def dscat_start(slot, b):
        pltpu.async_copy(wden2.at[b], den_sh.at[ei3.at[slot, 1]],
                         sem_s.at[b], add=True)

    def scat_start(slot, b):
        pltpu.async_copy(rows2.at[b], num_sh.at[ei3.at[slot, 1]],
                         sem_s.at[b], add=True)

    def scat_wait(slot, b):
        pltpu.make_async_copy(rows2.at[b], num_sh.at[ei3.at[slot, 1]],
                              sem_s.at[b]).wait()
        pltpu.make_async_copy(wden2.at[b], den_sh.at[ei3.at[slot, 1]],
                              sem_s.at[b]).wait()

    idx_start(0, 0)
    idx_wait(0)
    idx_start(1, 1)
    gath_start(0, 0)

    z16 = jnp.zeros((16,), jnp.int32)
    i16 = lax.iota(jnp.int32, 16)

    @pl.loop(0, _NCHUNK)
    def _chunk(g):
        p = lax.rem(g, 2)
        slot = lax.rem(g, 3)

        @pl.when(g > 0)
        def _():
            scat_wait(lax.rem(g + 2, 3), 1 - p)

        @pl.when(g < _NCHUNK - 1)
        def _():
            idx_wait(lax.rem(g + 1, 3))
            gath_start(lax.rem(g + 1, 3), 1 - p)

        @pl.when(g < _NCHUNK - 2)
        def _():
            idx_start(g + 2, lax.rem(g + 2, 3))

        alpha_wait(slot, p)
        for j in range(_C // 16):
            r16 = i16 + (j * 16)
            e = (plsc.load_gather(asr2.at[p], [r16, z16])
                 + plsc.load_gather(adr2.at[p], [r16, z16]))
            e = jnp.where(e > 0, e, e * 0.2)
            w16 = jnp.exp(e - m16)
            w_v[pl.ds(j * 16, 16)] = w16
            plsc.store_scatter(wden2.at[p], [r16, z16], w16)

        dscat_start(slot, p)

        rows_wait(slot, p)

        @plsc.parallel_loop(0, _C, 1, unroll=8)
        def _row(r):
            wb = plsc.load_gather(w_v, [jnp.full((16,), r, jnp.int32)])
            for c in range(8):
                sl = pl.ds(c * 16, 16)
                rows2[p, r, sl] = rows2[p, r, sl] * wb

        scat_start(slot, p)

    scat_wait(lax.rem(_NCHUNK - 1, 3), lax.rem(_NCHUNK - 1, 2))
    plsc.subcore_barrier()

    @pl.when(sid < _NDRAIN)
    def _():
        r0 = sid * _RPT
        pltpu.sync_copy(num_sh.at[pl.ds(r0, _RPT)],
                        num_out.at[cid, pl.ds(r0, _RPT)])
        pltpu.sync_copy(den_sh.at[pl.ds(r0, _RPT)],
                        den_out.at[cid, pl.ds(r0, _RPT)])


def _sc_edge_pass(h, edge_index, alpha_s, alpha_d, m_vec):
    mesh = plsc.VectorSubcoreMesh(core_axis_name="c", subcore_axis_name="s")
    f32 = jnp.float32
    k = pl.kernel(
        _sc_body,
        compiler_params=pltpu.CompilerParams(needs_layout_passes=False,
                                             use_tc_tiling_on_sc=False),
        out_type=(
            jax.ShapeDtypeStruct((_NC, N_NODES, HID_CH), f32),
            jax.ShapeDtypeStruct((_NC, N_NODES, 16), f32),
        ),
        mesh=mesh,
        scratch_types=[
            pltpu.VMEM((1, 16), f32),
            pltpu.VMEM((3, 2, _C), jnp.int32),
            pltpu.VMEM((2, _C, 16), f32),
            pltpu.VMEM((2, _C, 16), f32),
            pltpu.VMEM((2, _C, HID_CH), f32),
            pltpu.VMEM((2, _C, 16), f32),
            pltpu.VMEM((_C,), f32),
            pltpu.VMEM_SHARED((N_NODES, HID_CH), f32),
            pltpu.VMEM_SHARED((N_NODES, 16), f32),
            pltpu.SemaphoreType.DMA,
            pltpu.SemaphoreType.DMA((2,)),
            pltpu.SemaphoreType.DMA((2,)),
            pltpu.SemaphoreType.DMA((2,)),
        ],
    )
    return k(h, edge_index, alpha_s, alpha_d, m_vec)


def _decode_body(n0_ref, n1_ref, d0_ref, d1_ref, be_ref, wd_ref, bd_ref,
                 rec_ref, z_ref):
    num = n0_ref[0] + n1_ref[0]
    den = d0_ref[0, :, 0:1] + d1_ref[0, :, 0:1]
    z = num / (den + 1e-16) + be_ref[...]
    z_ref[...] = z
    rec_ref[...] = jnp.tanh(
        jnp.dot(z, wd_ref[...], preferred_element_type=jnp.float32)
        + bd_ref[...])


def _decode(num, den, b_enc, w_dec, b_dec):
    blk = 1000
    grid = (N_NODES // blk,)
    return pl.pallas_call(
        _decode_body,
        grid=grid,
        in_specs=[
            pl.BlockSpec((1, blk, HID_CH), lambda i: (0, i, 0)),
            pl.BlockSpec((1, blk, HID_CH), lambda i: (1, i, 0)),
            pl.BlockSpec((1, blk, 16), lambda i: (0, i, 0)),
            pl.BlockSpec((1, blk, 16), lambda i: (1, i, 0)),
            pl.BlockSpec((1, HID_CH), lambda i: (0, 0)),
            pl.BlockSpec((HID_CH, IN_CH), lambda i: (0, 0)),
            pl.BlockSpec((1, IN_CH), lambda i: (0, 0)),
        ],
        out_specs=[
            pl.BlockSpec((blk, IN_CH), lambda i: (i, 0)),
            pl.BlockSpec((blk, HID_CH), lambda i: (i, 0)),
        ],
        out_shape=[
            jax.ShapeDtypeStruct((N_NODES, IN_CH), jnp.float32),
            jax.ShapeDtypeStruct((N_NODES, HID_CH), jnp.float32),
        ],
    )(num, num, den, den, b_enc.reshape(1, HID_CH), w_dec,
      b_dec.reshape(1, IN_CH))


def kernel(x, edge_index, W_enc, a_src, a_dst, b_enc, W_dec, b_dec):
    h, als, ald, m_vec, _ = _encode(x, W_enc, a_src, a_dst)
    num, den = _sc_edge_pass(h, edge_index, als, ald, m_vec)
    recon, z = _decode(num, den, b_enc, W_dec, b_dec)
    return (recon, z)
